# baseline (speedup 1.0000x reference)
import functools

import jax
import jax.numpy as jnp
import numpy as np
from jax import lax
from jax.experimental import pallas as pl
from jax.experimental.pallas import tpu as pltpu
from jax.experimental.pallas import tpu_sc as plsc

_B, _C, _H, _W = 8, 3, 512, 512
_P = 16
_EMBED = 1024
_HEAD_DIM = 64
_HP = _H // _P
_WP = _W // _P
_N = _B * _HP * _WP
_K = _C * _P * _P
_M_BLK = _HP * _WP

_B_TC = 3
_B_SC = _B - _B_TC

_KP = _C * _P
_ROWS = _B_SC * _HP * _WP * _KP
_NC, _NS = 2, 16
_NW = _NC * _NS
_PER_W = _ROWS // _NW
_NCHUNK = 3
_CH = _PER_W // _NCHUNK


def _gather_index():
    r = jnp.arange(_ROWS, dtype=jnp.int32)
    t, k = r // _KP + _B_TC * _HP * _WP, r % _KP
    b, hw = t // (_HP * _WP), t % (_HP * _WP)
    hy, wx = hw // _WP, hw % _WP
    c, py = k // _P, k % _P
    return ((b * _C + c) * _H + hy * _P + py) * _WP + wx


def _sc_patchify(table, idx):
    mesh = plsc.VectorSubcoreMesh(core_axis_name="c", subcore_axis_name="s")

    @functools.partial(
        pl.kernel,
        mesh=mesh,
        compiler_params=pltpu.CompilerParams(use_tc_tiling_on_sc=False),
        out_type=jax.ShapeDtypeStruct((_ROWS, _P), jnp.float32),
        scratch_types=[
            pltpu.VMEM((_CH,), jnp.int32),
            pltpu.VMEM((_CH, _P), jnp.float32),
            pltpu.SemaphoreType.DMA,
        ],
    )
    def gather_kernel(table_hbm, idx_hbm, out_hbm, idx_v, rows_v, sem):
        wid = lax.axis_index("s") * _NC + lax.axis_index("c")
        base = wid * _PER_W
        for j in range(_NCHUNK):
            off = base + j * _CH
            pltpu.sync_copy(idx_hbm.at[pl.ds(off, _CH)], idx_v)
            pltpu.async_copy(table_hbm.at[idx_v], rows_v, sem).wait()
            pltpu.sync_copy(rows_v, out_hbm.at[pl.ds(off, _CH)])

    return gather_kernel(table, idx)


def _fused_body(img_ref, w_ref, b_ref, o_ref):
    a = img_ref[0].astype(jnp.bfloat16)
    a = a.reshape(_C, _HP, _P, _WP, _P)
    a = a.transpose(1, 3, 0, 2, 4)
    a = a.reshape(_M_BLK, _K)
    acc = jax.lax.dot_general(
        a, w_ref[...],
        (((1,), (1,)), ((), ())),
        preferred_element_type=jnp.float32,
    )
    o_ref[...] = acc + b_ref[...]


def _project_tc(images, w_bf16, bias_row):
    img5 = images.reshape(_B, _C, _HP, _P, _W)
    return pl.pallas_call(
        _fused_body,
        grid=(_B_TC,),
        in_specs=[
            pl.BlockSpec((1, _C, _HP, _P, _W), lambda m: (m, 0, 0, 0, 0)),
            pl.BlockSpec((_EMBED, _K), lambda m: (0, 0)),
            pl.BlockSpec((1, _EMBED), lambda m: (0, 0)),
        ],
        out_specs=pl.BlockSpec((_M_BLK, _EMBED), lambda m: (m, 0)),
        out_shape=jax.ShapeDtypeStruct((_N, _EMBED), jnp.float32),
    )(img5, w_bf16, bias_row)


def _gemm_body(a_ref, w_ref, b_ref, tok_ref, o_ref):
    a = a_ref[...].astype(jnp.bfloat16)
    acc = jax.lax.dot_general(
        a, w_ref[...],
        (((1,), (1,)), ((), ())),
        preferred_element_type=jnp.float32,
    )
    o_ref[...] = acc + b_ref[...]


def _project_sc_rows(raw_hi, w_bf16, bias_row, tokens_partial):
    return pl.pallas_call(
        _gemm_body,
        grid=(_B_SC,),
        in_specs=[
            pl.BlockSpec((_M_BLK, _K), lambda m: (m, 0)),
            pl.BlockSpec((_EMBED, _K), lambda m: (0, 0)),
            pl.BlockSpec((1, _EMBED), lambda m: (0, 0)),
            pl.BlockSpec(memory_space=pltpu.MemorySpace.HBM),
        ],
        out_specs=pl.BlockSpec((_M_BLK, _EMBED), lambda m: (m + _B_TC, 0)),
        out_shape=jax.ShapeDtypeStruct((_N, _EMBED), jnp.float32),
        input_output_aliases={3: 0},
    )(raw_hi, w_bf16, bias_row, tokens_partial)


def _side_outputs():
    ys, xs = jnp.meshgrid(jnp.arange(_HP), jnp.arange(_WP), indexing="ij")
    coords = jnp.stack([ys, xs], axis=-1).reshape(-1, 2)
    patch_coords = jnp.tile(coords, (_B, 1))
    d_axis = _HEAD_DIM // 2
    n_freq = d_axis // 2
    inv_freq = 1.0 / (10000.0 ** (jnp.arange(n_freq, dtype=jnp.float32) / n_freq))
    cf = patch_coords.astype(jnp.float32)
    ang_y = cf[:, 0:1] * inv_freq[None, :]
    ang_x = cf[:, 1:2] * inv_freq[None, :]
    ang = jnp.concatenate([ang_y, ang_x], axis=-1)
    emb = jnp.concatenate([ang, ang], axis=-1)
    rope_cos, rope_sin = jnp.cos(emb), jnp.sin(emb)
    cu_seqlens = jnp.arange(_B + 1, dtype=jnp.int32) * (_HP * _WP)
    is_patch = jnp.ones((_N,), dtype=jnp.bool_)
    return cu_seqlens, patch_coords, rope_cos, rope_sin, is_patch


def kernel(images, W, b):
    w_bf = W.astype(jnp.bfloat16)
    bias_row = b.reshape(1, _EMBED)
    table = images.reshape(_B * _C * _H * _WP, _P)
    raw_hi = _sc_patchify(table, _gather_index())
    tokens_partial = _project_tc(images, w_bf, bias_row)
    tokens = _project_sc_rows(raw_hi, w_bf, bias_row, tokens_partial)
    cu_seqlens, patch_coords, rope_cos, rope_sin, is_patch = _side_outputs()
    return tokens, cu_seqlens, patch_coords, rope_cos, rope_sin, is_patch


# speedup vs baseline: 1.4060x; 1.4060x over previous
import functools

import jax
import jax.numpy as jnp
import numpy as np
from jax import lax
from jax.experimental import pallas as pl
from jax.experimental.pallas import tpu as pltpu
from jax.experimental.pallas import tpu_sc as plsc

_B, _C, _H, _W = 8, 3, 512, 512
_P = 16
_EMBED = 1024
_HEAD_DIM = 64
_HP = _H // _P
_WP = _W // _P
_N = _B * _HP * _WP
_K = _C * _P * _P
_M_BLK = _HP * _WP

_B_TC = 3
_B_SC = _B - _B_TC

_KP = _C * _P
_ROWS = _B_SC * _HP * _WP * _KP
_NC, _NS = 2, 16
_NW = _NC * _NS
_PER_W = _ROWS // _NW
_NCHUNK = 3
_CH = _PER_W // _NCHUNK


def _gather_index():
    r = jnp.arange(_ROWS, dtype=jnp.int32)
    t, k = r // _KP + _B_TC * _HP * _WP, r % _KP
    b, hw = t // (_HP * _WP), t % (_HP * _WP)
    hy, wx = hw // _WP, hw % _WP
    c, py = k // _P, k % _P
    return ((b * _C + c) * _H + hy * _P + py) * _WP + wx


def _sc_patchify(table, idx):
    mesh = plsc.VectorSubcoreMesh(core_axis_name="c", subcore_axis_name="s")

    @functools.partial(
        pl.kernel,
        mesh=mesh,
        compiler_params=pltpu.CompilerParams(use_tc_tiling_on_sc=False),
        out_type=jax.ShapeDtypeStruct((_ROWS, _P), jnp.float32),
        scratch_types=[
            pltpu.VMEM((_CH,), jnp.int32),
            pltpu.VMEM((_CH, _P), jnp.float32),
            pltpu.SemaphoreType.DMA,
        ],
    )
    def gather_kernel(table_hbm, idx_hbm, out_hbm, idx_v, rows_v, sem):
        wid = lax.axis_index("s") * _NC + lax.axis_index("c")
        base = wid * _PER_W
        for j in range(_NCHUNK):
            off = base + j * _CH
            pltpu.sync_copy(idx_hbm.at[pl.ds(off, _CH)], idx_v)
            pltpu.async_copy(table_hbm.at[idx_v], rows_v, sem).wait()
            pltpu.sync_copy(rows_v, out_hbm.at[pl.ds(off, _CH)])

    return gather_kernel(table, idx)


def _fused_body(img_ref, w_ref, b_ref, o_ref):
    a = img_ref[0].astype(jnp.bfloat16)
    a = a.reshape(_C, _HP, _P, _WP, _P)
    a = a.transpose(1, 3, 0, 2, 4)
    a = a.reshape(_M_BLK, _K)
    acc = jax.lax.dot_general(
        a, w_ref[...],
        (((1,), (1,)), ((), ())),
        preferred_element_type=jnp.float32,
    )
    o_ref[...] = acc + b_ref[...]


def _project_tc(images, w_bf16, bias_row):
    img5 = images.reshape(_B, _C, _HP, _P, _W)
    return pl.pallas_call(
        _fused_body,
        grid=(_B_TC,),
        in_specs=[
            pl.BlockSpec((1, _C, _HP, _P, _W), lambda m: (m, 0, 0, 0, 0)),
            pl.BlockSpec((_EMBED, _K), lambda m: (0, 0)),
            pl.BlockSpec((1, _EMBED), lambda m: (0, 0)),
        ],
        out_specs=pl.BlockSpec((_M_BLK, _EMBED), lambda m: (m, 0)),
        out_shape=jax.ShapeDtypeStruct((_N, _EMBED), jnp.float32),
    )(img5, w_bf16, bias_row)


def _gemm_body(a_ref, w_ref, b_ref, tok_ref, o_ref):
    a = a_ref[...].astype(jnp.bfloat16)
    acc = jax.lax.dot_general(
        a, w_ref[...],
        (((1,), (1,)), ((), ())),
        preferred_element_type=jnp.float32,
    )
    o_ref[...] = acc + b_ref[...]


def _project_sc_rows(raw_hi, w_bf16, bias_row, tokens_partial):
    return pl.pallas_call(
        _gemm_body,
        grid=(_B_SC,),
        in_specs=[
            pl.BlockSpec((_M_BLK, _K), lambda m: (m, 0)),
            pl.BlockSpec((_EMBED, _K), lambda m: (0, 0)),
            pl.BlockSpec((1, _EMBED), lambda m: (0, 0)),
            pl.BlockSpec(memory_space=pltpu.MemorySpace.HBM),
        ],
        out_specs=pl.BlockSpec((_M_BLK, _EMBED), lambda m: (m + _B_TC, 0)),
        out_shape=jax.ShapeDtypeStruct((_N, _EMBED), jnp.float32),
        input_output_aliases={3: 0},
    )(raw_hi, w_bf16, bias_row, tokens_partial)


def _side_outputs():
    ys, xs = jnp.meshgrid(jnp.arange(_HP), jnp.arange(_WP), indexing="ij")
    coords = jnp.stack([ys, xs], axis=-1).reshape(-1, 2)
    patch_coords = jnp.tile(coords, (_B, 1))
    d_axis = _HEAD_DIM // 2
    n_freq = d_axis // 2
    inv_freq = 1.0 / (10000.0 ** (jnp.arange(n_freq, dtype=jnp.float32) / n_freq))
    cf = patch_coords.astype(jnp.float32)
    ang_y = cf[:, 0:1] * inv_freq[None, :]
    ang_x = cf[:, 1:2] * inv_freq[None, :]
    ang = jnp.concatenate([ang_y, ang_x], axis=-1)
    emb = jnp.concatenate([ang, ang], axis=-1)
    rope_cos, rope_sin = jnp.cos(emb), jnp.sin(emb)
    cu_seqlens = jnp.arange(_B + 1, dtype=jnp.int32) * (_HP * _WP)
    is_patch = jnp.ones((_N,), dtype=jnp.bool_)
    return cu_seqlens, patch_coords, rope_cos, rope_sin, is_patch


def kernel(images, W, b):
    w_bf = W.astype(jnp.bfloat16)
    bias_row = b.reshape(1, _EMBED)
    table = images.reshape(_B * _C * _H * _WP, _P)
    raw_hi = _sc_patchify(table, _gather_index())
    raw_hi = raw_hi.reshape(_B_SC * _M_BLK, _K)
    tokens_partial = _project_tc(images, w_bf, bias_row)
    tokens = _project_sc_rows(raw_hi, w_bf, bias_row, tokens_partial)
    cu_seqlens, patch_coords, rope_cos, rope_sin, is_patch = _side_outputs()
    return tokens, cu_seqlens, patch_coords, rope_cos, rope_sin, is_patch


# speedup vs baseline: 1.5406x; 1.0957x over previous
import jax
import jax.numpy as jnp
import numpy as np
from jax.experimental import pallas as pl
from jax.experimental.pallas import tpu as pltpu

_B, _C, _H, _W = 8, 3, 512, 512
_P = 16
_EMBED = 1024
_HEAD_DIM = 64
_HP = _H // _P
_WP = _W // _P
_N = _B * _HP * _WP
_K = _C * _P * _P
_M_BLK = _HP * _WP


def _fused_body(img_ref, w_ref, b_ref, o_ref):
    a = img_ref[0].astype(jnp.bfloat16)
    a = a.reshape(_C, _HP, _P, _WP, _P)
    a = a.transpose(1, 3, 0, 2, 4)
    a = a.reshape(_M_BLK, _K)
    acc = jax.lax.dot_general(
        a, w_ref[...],
        (((1,), (1,)), ((), ())),
        preferred_element_type=jnp.float32,
    )
    o_ref[...] = acc + b_ref[...]


def _project(images, w_bf16, bias_row):
    img5 = images.reshape(_B, _C, _HP, _P, _W)
    return pl.pallas_call(
        _fused_body,
        grid=(_B,),
        in_specs=[
            pl.BlockSpec((1, _C, _HP, _P, _W), lambda m: (m, 0, 0, 0, 0)),
            pl.BlockSpec((_EMBED, _K), lambda m: (0, 0)),
            pl.BlockSpec((1, _EMBED), lambda m: (0, 0)),
        ],
        out_specs=pl.BlockSpec((_M_BLK, _EMBED), lambda m: (m, 0)),
        out_shape=jax.ShapeDtypeStruct((_N, _EMBED), jnp.float32),
    )(img5, w_bf16, bias_row)


def _side_outputs():
    ys, xs = jnp.meshgrid(jnp.arange(_HP), jnp.arange(_WP), indexing="ij")
    coords = jnp.stack([ys, xs], axis=-1).reshape(-1, 2)
    patch_coords = jnp.tile(coords, (_B, 1))
    d_axis = _HEAD_DIM // 2
    n_freq = d_axis // 2
    inv_freq = 1.0 / (10000.0 ** (jnp.arange(n_freq, dtype=jnp.float32) / n_freq))
    cf = patch_coords.astype(jnp.float32)
    ang_y = cf[:, 0:1] * inv_freq[None, :]
    ang_x = cf[:, 1:2] * inv_freq[None, :]
    ang = jnp.concatenate([ang_y, ang_x], axis=-1)
    emb = jnp.concatenate([ang, ang], axis=-1)
    rope_cos, rope_sin = jnp.cos(emb), jnp.sin(emb)
    cu_seqlens = jnp.arange(_B + 1, dtype=jnp.int32) * (_HP * _WP)
    is_patch = jnp.ones((_N,), dtype=jnp.bool_)
    return cu_seqlens, patch_coords, rope_cos, rope_sin, is_patch


def kernel(images, W, b):
    w_bf = W.astype(jnp.bfloat16)
    tokens = _project(images, w_bf, b.reshape(1, _EMBED))
    cu_seqlens, patch_coords, rope_cos, rope_sin, is_patch = _side_outputs()
    return tokens, cu_seqlens, patch_coords, rope_cos, rope_sin, is_patch


# speedup vs baseline: 1.5451x; 1.0029x over previous
import jax
import jax.numpy as jnp
import numpy as np
from jax.experimental import pallas as pl
from jax.experimental.pallas import tpu as pltpu

_B, _C, _H, _W = 8, 3, 512, 512
_P = 16
_EMBED = 1024
_HEAD_DIM = 64
_HP = _H // _P
_WP = _W // _P
_N = _B * _HP * _WP
_K = _C * _P * _P
_M_BLK = _HP * _WP


def _fused_body(img_ref, w_ref, b_ref, o_ref):
    _Q = 4
    hq = _HP // _Q
    for q in range(_Q):
        a = img_ref[0, :, q * hq:(q + 1) * hq].astype(jnp.bfloat16)
        a = a.reshape(_C, hq, _P, _WP, _P)
        a = a.transpose(1, 3, 0, 2, 4)
        a = a.reshape(hq * _WP, _K)
        acc = jax.lax.dot_general(
            a, w_ref[...],
            (((1,), (1,)), ((), ())),
            preferred_element_type=jnp.float32,
        )
        o_ref[q * hq * _WP:(q + 1) * hq * _WP] = acc + b_ref[...]


def _project(images, w_bf16, bias_row):
    img5 = images.reshape(_B, _C, _HP, _P, _W)
    return pl.pallas_call(
        _fused_body,
        grid=(_B,),
        in_specs=[
            pl.BlockSpec((1, _C, _HP, _P, _W), lambda m: (m, 0, 0, 0, 0)),
            pl.BlockSpec((_EMBED, _K), lambda m: (0, 0)),
            pl.BlockSpec((1, _EMBED), lambda m: (0, 0)),
        ],
        out_specs=pl.BlockSpec((_M_BLK, _EMBED), lambda m: (m, 0)),
        out_shape=jax.ShapeDtypeStruct((_N, _EMBED), jnp.float32),
    )(img5, w_bf16, bias_row)


def _side_outputs():
    ys, xs = jnp.meshgrid(jnp.arange(_HP), jnp.arange(_WP), indexing="ij")
    coords = jnp.stack([ys, xs], axis=-1).reshape(-1, 2)
    patch_coords = jnp.tile(coords, (_B, 1))
    d_axis = _HEAD_DIM // 2
    n_freq = d_axis // 2
    inv_freq = 1.0 / (10000.0 ** (jnp.arange(n_freq, dtype=jnp.float32) / n_freq))
    cf = patch_coords.astype(jnp.float32)
    ang_y = cf[:, 0:1] * inv_freq[None, :]
    ang_x = cf[:, 1:2] * inv_freq[None, :]
    ang = jnp.concatenate([ang_y, ang_x], axis=-1)
    emb = jnp.concatenate([ang, ang], axis=-1)
    rope_cos, rope_sin = jnp.cos(emb), jnp.sin(emb)
    cu_seqlens = jnp.arange(_B + 1, dtype=jnp.int32) * (_HP * _WP)
    is_patch = jnp.ones((_N,), dtype=jnp.bool_)
    return cu_seqlens, patch_coords, rope_cos, rope_sin, is_patch


def kernel(images, W, b):
    w_bf = W.astype(jnp.bfloat16)
    tokens = _project(images, w_bf, b.reshape(1, _EMBED))
    cu_seqlens, patch_coords, rope_cos, rope_sin, is_patch = _side_outputs()
    return tokens, cu_seqlens, patch_coords, rope_cos, rope_sin, is_patch
